# baseline (speedup 1.0000x reference)
import functools

import jax
import jax.numpy as jnp
from jax import lax
from jax.experimental import pallas as pl
from jax.experimental.pallas import tpu as pltpu
from jax.experimental.pallas import tpu_sc as plsc

B = 64
S = 512
D = 2048
K = 8192
KT = 1024
NKT = K // KT
RB = 2
SC_ROWS = 16
TC_ROWS = B - SC_ROWS
P1 = TC_ROWS // RB
P2 = NKT
P3 = B // RB
ALPHA_BASE = 0.3
MAX_DELTA = 0.5

HALF = D // 2
CH = 32
NCH = S // CH
LG = HALF // 16


def _sc_mean_kernel(h_hbm, out_hbm, buf0, buf1, acc_ref, obuf,
                    sem0, sem1, osem):
    c = lax.axis_index("c")
    s = lax.axis_index("s")
    wid = s * 2 + c
    row = TC_ROWS + wid // 2
    half = wid % 2
    d0 = half * HALF

    def chunk_src(k):
        return h_hbm.at[row, pl.ds(k * CH, CH), pl.ds(d0, HALF)]

    def zero_body(g, _):
        acc_ref[pl.ds(g * 16, 16)] = jnp.zeros((16,), jnp.float32)
        return 0
    lax.fori_loop(0, LG, zero_body, 0)

    pltpu.make_async_copy(chunk_src(0), buf0, sem0).start()

    def accumulate(buf):
        def group_body(g, _):
            sl = pl.ds(g * 16, 16)

            def row_body(r, a):
                return a + buf[r, sl]
            acc_ref[sl] = lax.fori_loop(0, CH, row_body, acc_ref[sl],
                                        unroll=8)
            return 0
        lax.fori_loop(0, LG, group_body, 0)

    def two_chunks(kk, _):
        k = kk * 2
        pltpu.make_async_copy(chunk_src(k), buf0, sem0).wait()

        @pl.when(k + 1 < NCH)
        def _pf1():
            pltpu.make_async_copy(chunk_src(k + 1), buf1, sem1).start()

        accumulate(buf0)
        pltpu.make_async_copy(chunk_src(k + 1), buf1, sem1).wait()

        @pl.when(k + 2 < NCH)
        def _pf2():
            pltpu.make_async_copy(chunk_src(k + 2), buf0, sem0).start()

        accumulate(buf1)
        return 0

    lax.fori_loop(0, NCH // 2, two_chunks, 0)

    inv = jnp.full((16,), 1.0 / S, jnp.float32)

    def scale_body(g, _):
        sl = pl.ds(g * 16, 16)
        obuf[sl] = acc_ref[sl] * inv
        return 0
    lax.fori_loop(0, LG, scale_body, 0)

    out_dst = out_hbm.at[wid // 2, pl.ds(d0, HALF)]
    pltpu.make_async_copy(obuf, out_dst, osem).start()
    pltpu.make_async_copy(obuf, out_dst, osem).wait()


_sc_mean = functools.partial(
    pl.kernel,
    out_type=jax.ShapeDtypeStruct((SC_ROWS, D), jnp.float32),
    mesh=plsc.VectorSubcoreMesh(core_axis_name="c", subcore_axis_name="s"),
    scratch_types=[
        pltpu.VMEM((CH, HALF), jnp.float32),
        pltpu.VMEM((CH, HALF), jnp.float32),
        pltpu.VMEM((HALF,), jnp.float32),
        pltpu.VMEM((HALF,), jnp.float32),
        pltpu.SemaphoreType.DMA,
        pltpu.SemaphoreType.DMA,
        pltpu.SemaphoreType.DMA,
    ],
)(_sc_mean_kernel)


def _tc_mean_kernel(h_ref, o_ref):
    o_ref[...] = jnp.mean(h_ref[...], axis=1, keepdims=True)


def _tc_main_kernel(vraw48_ref, vsc_ref, a_ref, h_ref, o_ref,
                    vraw_ref, vnorm_ref, best_ref, rmax_ref, vdiff_ref):
    i = pl.program_id(0)

    @pl.when(i < P2)
    def _phase_lookup():
        j = i

        @pl.when(j == 0)
        def _init():
            vraw_ref[pl.ds(0, TC_ROWS), :] = vraw48_ref[...]
            vraw_ref[pl.ds(TC_ROWS, SC_ROWS), :] = vsc_ref[...]
            v = vraw_ref[...]
            bm = jnp.mean(v, axis=0)
            bv = jnp.mean((v - bm[None, :]) ** 2, axis=0)
            vnorm_ref[...] = (v - bm[None, :]) / jnp.sqrt(bv + 1e-8)[None, :]
            rmax_ref[...] = jnp.full((B, 128), -jnp.inf, jnp.float32)
            best_ref[...] = jnp.zeros((B, D), jnp.float32)

        a = a_ref[...]
        rn = 1.0 / jnp.maximum(jnp.sqrt(jnp.sum(a * a, axis=1)), 1e-8)
        vn = vnorm_ref[...]
        cos = jax.lax.dot_general(
            vn, a, (((1,), (1,)), ((), ())),
            preferred_element_type=jnp.float32)
        cos = cos * rn[None, :]
        tile_max = jnp.max(cos, axis=1)
        tile_arg = jnp.argmax(cos, axis=1)
        run_max = rmax_ref[:, 0]
        improved = tile_max > run_max
        onehot = jnp.where(
            jax.lax.broadcasted_iota(jnp.int32, (B, KT), 1) == tile_arg[:, None],
            rn[None, :], 0.0)
        cand = jax.lax.dot_general(
            onehot, a, (((1,), (0,)), ((), ())),
            preferred_element_type=jnp.float32)
        best_ref[...] = jnp.where(improved[:, None], cand, best_ref[...])
        new_max = jnp.where(improved, tile_max, run_max)
        rmax_ref[...] = jnp.broadcast_to(new_max[:, None], (B, 128))

        @pl.when(j == P2 - 1)
        def _finish():
            vnorm = vnorm_ref[...]
            score = rmax_ref[:, 0]
            alpha = ALPHA_BASE * (1.0 - score)
            delta = jnp.clip(best_ref[...] - vnorm, -MAX_DELTA, MAX_DELTA)
            v_snapped = vnorm + alpha[:, None] * delta
            vdiff_ref[...] = v_snapped - vraw_ref[...]

    @pl.when(i >= P2)
    def _phase_add():
        b = i - P2
        rows = [vdiff_ref[pl.ds(b * RB + r, 1), :] for r in range(RB)]
        o_ref[...] = h_ref[...] + jnp.concatenate(rows, axis=0)[:, None, :]


def _main_h_index(i):
    return (jnp.where(i < P2, 0, i - P2), 0, 0)


def _main_o_index(i):
    return (jnp.where(i < P2, 0, i - P2), 0, 0)


def _main_a_index(i):
    return (jnp.clip(i, 0, P2 - 1), 0)


@jax.jit
def kernel(hidden_states, attractors):
    v_sc = _sc_mean(hidden_states)

    vraw48 = pl.pallas_call(
        _tc_mean_kernel,
        grid=(P1,),
        in_specs=[pl.BlockSpec((RB, S, D), lambda i: (i, 0, 0))],
        out_specs=pl.BlockSpec((RB, 1, D), lambda i: (i, 0, 0)),
        out_shape=jax.ShapeDtypeStruct((TC_ROWS, 1, D), jnp.float32),
    )(hidden_states)

    out = pl.pallas_call(
        _tc_main_kernel,
        grid=(P2 + P3,),
        in_specs=[
            pl.BlockSpec((TC_ROWS, D), lambda i: (0, 0)),
            pl.BlockSpec((SC_ROWS, D), lambda i: (0, 0)),
            pl.BlockSpec((KT, D), _main_a_index),
            pl.BlockSpec((RB, S, D), _main_h_index),
        ],
        out_specs=pl.BlockSpec((RB, S, D), _main_o_index),
        out_shape=jax.ShapeDtypeStruct((B, S, D), jnp.float32),
        scratch_shapes=[
            pltpu.VMEM((B, D), jnp.float32),
            pltpu.VMEM((B, D), jnp.float32),
            pltpu.VMEM((B, D), jnp.float32),
            pltpu.VMEM((B, 128), jnp.float32),
            pltpu.VMEM((B, D), jnp.float32),
        ],
    )(vraw48.reshape(TC_ROWS, D), v_sc, attractors, hidden_states)
    return out


# speedup vs baseline: 1.0948x; 1.0948x over previous
import jax
import jax.numpy as jnp
from jax.experimental import pallas as pl
from jax.experimental.pallas import tpu as pltpu

B = 64
S = 512
D = 2048
K = 8192
KT = 1024
NKT = K // KT
RB = 2
P1 = B // RB
P2 = NKT
ALPHA_BASE = 0.3
MAX_DELTA = 0.5


def _fused_kernel(h_ref, a_ref, o_ref, vraw_ref, vnorm_ref, best_ref, rmax_ref,
                  vdiff_ref):
    i = pl.program_id(0)

    @pl.when(i < P1)
    def _phase_mean():
        m = jnp.mean(h_ref[...], axis=1)
        for r in range(RB):
            vraw_ref[pl.ds(i * RB + r, 1), :] = m[r][None, :]

    @pl.when(jnp.logical_and(i >= P1, i < P1 + P2))
    def _phase_lookup():
        j = i - P1

        @pl.when(j == 0)
        def _init():
            v = vraw_ref[...]
            bm = jnp.mean(v, axis=0)
            bv = jnp.mean((v - bm[None, :]) ** 2, axis=0)
            vnorm_ref[...] = (v - bm[None, :]) / jnp.sqrt(bv + 1e-8)[None, :]
            rmax_ref[...] = jnp.full((B, 128), -jnp.inf, jnp.float32)
            best_ref[...] = jnp.zeros((B, D), jnp.float32)

        a = a_ref[...]
        rn = 1.0 / jnp.maximum(jnp.sqrt(jnp.sum(a * a, axis=1)), 1e-8)
        vn = vnorm_ref[...]
        cos = jax.lax.dot_general(
            vn, a, (((1,), (1,)), ((), ())),
            preferred_element_type=jnp.float32)
        cos = cos * rn[None, :]
        tile_max = jnp.max(cos, axis=1)
        tile_arg = jnp.argmax(cos, axis=1)
        run_max = rmax_ref[:, 0]
        improved = tile_max > run_max
        onehot = jnp.where(
            jax.lax.broadcasted_iota(jnp.int32, (B, KT), 1) == tile_arg[:, None],
            rn[None, :], 0.0)
        cand = jax.lax.dot_general(
            onehot, a, (((1,), (0,)), ((), ())),
            preferred_element_type=jnp.float32)
        best_ref[...] = jnp.where(improved[:, None], cand, best_ref[...])
        new_max = jnp.where(improved, tile_max, run_max)
        rmax_ref[...] = jnp.broadcast_to(new_max[:, None], (B, 128))

        @pl.when(j == P2 - 1)
        def _finish():
            vnorm = vnorm_ref[...]
            score = rmax_ref[:, 0]
            alpha = ALPHA_BASE * (1.0 - score)
            delta = jnp.clip(best_ref[...] - vnorm, -MAX_DELTA, MAX_DELTA)
            v_snapped = vnorm + alpha[:, None] * delta
            vdiff_ref[...] = v_snapped - vraw_ref[...]

    @pl.when(i >= P1 + P2)
    def _phase_add():
        b = (P1 - 1) - (i - (P1 + P2))
        rows = [vdiff_ref[pl.ds(b * RB + r, 1), :] for r in range(RB)]
        o_ref[...] = h_ref[...] + jnp.concatenate(rows, axis=0)[:, None, :]


def _h_index(i):
    b = jnp.where(i < P1, i,
                  jnp.where(i < P1 + P2, P1 - 1, (P1 - 1) - (i - (P1 + P2))))
    return (b, 0, 0)


def _a_index(i):
    j = jnp.clip(i - P1, 0, P2 - 1)
    return (j, 0)


def _o_index(i):
    b = jnp.where(i < P1 + P2, P1 - 1, (P1 - 1) - (i - (P1 + P2)))
    return (b, 0, 0)


@jax.jit
def kernel(hidden_states, attractors):
    return pl.pallas_call(
        _fused_kernel,
        grid=(P1 + P2 + P1,),
        in_specs=[
            pl.BlockSpec((RB, S, D), _h_index),
            pl.BlockSpec((KT, D), _a_index),
        ],
        out_specs=pl.BlockSpec((RB, S, D), _o_index),
        out_shape=jax.ShapeDtypeStruct((B, S, D), jnp.float32),
        scratch_shapes=[
            pltpu.VMEM((B, D), jnp.float32),
            pltpu.VMEM((B, D), jnp.float32),
            pltpu.VMEM((B, D), jnp.float32),
            pltpu.VMEM((B, 128), jnp.float32),
            pltpu.VMEM((B, D), jnp.float32),
        ],
    )(hidden_states, attractors)
